# baseline (speedup 1.0000x reference)
import functools

import functools

import jax
import jax.numpy as jnp
import numpy as np
from jax import lax
from jax.experimental import pallas as pl
from jax.experimental.pallas import tpu as pltpu
from jax.experimental.pallas import tpu_sc as plsc

PAST_LEN = 8
FUTURE_LEN = 1
DIM = 64
N_MEM = 16384
TOPK = 200
NCLUSTER = 20
KM_ITER = 10
B = 1024


def _normalize(x, eps=1e-12):
    n = jnp.linalg.norm(x, axis=1, keepdims=True)
    return x / jnp.maximum(n, eps)


def _compute_kmeans_perms():
    key = jax.random.key(123)
    keys = jax.random.split(key, KM_ITER + 1)
    outs = []
    for i in range(KM_ITER + 1):
        pki = jax.random.split(keys[i], B)
        perm = jax.vmap(lambda k: jax.random.permutation(k, TOPK))(pki)
        outs.append(np.asarray(perm[:, :NCLUSTER]))
    return np.stack(outs)


try:
    with jax.default_device(jax.devices("cpu")[0]):
        _KM_PERMS = _compute_kmeans_perms()
except Exception:
    try:
        _KM_PERMS = _compute_kmeans_perms()
    except Exception:
        _KM_PERMS = None


def _kmeans_perms():
    if _KM_PERMS is not None:
        return _KM_PERMS
    key = jax.random.key(123)
    keys = jax.random.split(key, KM_ITER + 1)
    outs = []
    for i in range(KM_ITER + 1):
        pki = jax.random.split(keys[i], B)
        perm = jax.vmap(lambda k: jax.random.permutation(k, TOPK))(pki)
        outs.append(perm[:, :NCLUSTER])
    return jnp.stack(outs)


def _kmeans(batch_x, ncluster=NCLUSTER, niter=KM_ITER):
    b, n, d = batch_x.shape
    perms = _kmeans_perms()
    idx0 = jnp.broadcast_to(jnp.asarray(perms[0])[:, :, None], (b, ncluster, d))
    c = jnp.take_along_axis(batch_x, idx0, axis=1)
    for it in range(niter):
        d2 = ((batch_x[:, :, None, :] - c[:, None, :, :]) ** 2).sum(-1)
        a = jnp.argmin(d2, axis=2)
        onehot = jax.nn.one_hot(a, ncluster, dtype=batch_x.dtype)
        counts = onehot.sum(1)
        sums = jnp.einsum('bnk,bnd->bkd', onehot, batch_x)
        cnew = sums / jnp.maximum(counts, 1e-9)[:, :, None]
        dead = counts < 0.5
        idxi = jnp.broadcast_to(jnp.asarray(perms[it + 1])[:, :, None], (b, ncluster, d))
        repl = jnp.take_along_axis(batch_x, idxi, axis=1)
        c = jnp.where(dead[:, :, None], repl, cnew)
    return c


def _identity_kernel(x_ref, o_ref):
    o_ref[...] = x_ref[...]


_SC_NC = 2
_SC_NS = 16
_NW = _SC_NC * _SC_NS
_ROWS_PER_W = B // _NW
C_CAP = 256
_NV = N_MEM // 16
_U = 8


def _keyify(x):
    ui = lax.bitcast_convert_type(x, jnp.int32)
    flip = (ui >> 31) | jnp.int32(-2147483648)
    return lax.bitcast_convert_type(ui ^ flip, jnp.uint32)


def _desc_cum(hist2_ref, cum_ref, zero16):
    carry = zero16
    for j in range(15, -1, -1):
        hv = zero16
        for l in range(16):
            hv = hv + hist2_ref[pl.ds(l * 256 + 16 * j, 16)]
        suf = plsc.cumsum(lax.rev(hv, (0,))) + carry
        cum_ref[pl.ds(16 * j, 16)] = lax.rev(suf, (0,))
        carry = carry + jnp.sum(hv)
    cum_ref[pl.ds(256, 16)] = zero16


def _find_bin(cum_ref, above, zero16, k):
    acc = zero16
    for j in range(16):
        c = cum_ref[pl.ds(16 * j, 16)]
        acc = acc + plsc.all_reduce_population_count((c + above) >= k)
    return acc - 1


def _sc_topk_body(w_hbm, vals_hbm, cols_hbm,
                  wrow, keybuf, hist2, fhist2, ccum, fcum, cvals, cidx):
    wid = lax.axis_index("s") * _SC_NC + lax.axis_index("c")
    base_row = wid * _ROWS_PER_W
    zero16 = jnp.zeros((16,), jnp.int32)
    ones16 = jnp.ones((16,), jnp.int32)
    iota16 = lax.iota(jnp.int32, 16)
    lane_base = iota16 * 256
    neginf16 = jnp.full((16,), -jnp.inf, jnp.float32)

    def row_body(r, carry0):
        row = base_row + r
        pltpu.sync_copy(w_hbm.at[row], wrow)

        def zero_hists(j, c):
            for t in range(8):
                hist2[pl.ds((j * 8 + t) * 16, 16)] = zero16
                fhist2[pl.ds((j * 8 + t) * 16, 16)] = zero16
            return c
        lax.fori_loop(0, 32, zero_hists, 0)
        for j in range(C_CAP // 16):
            cvals[pl.ds(j * 16, 16)] = neginf16
            cidx[pl.ds(j * 16, 16)] = zero16

        def scan_a(i, c):
            for t in range(_U):
                ii = i * _U + t
                k = _keyify(wrow[pl.ds(ii * 16, 16)])
                keybuf[pl.ds(ii * 16, 16)] = k
                c8 = lax.convert_element_type(k >> jnp.uint32(24), jnp.int32)
                plsc.addupdate_scatter(hist2, [lane_base + c8], ones16,
                                       mask=jnp.full((16,), True))
            return c
        lax.fori_loop(0, _NV // _U, scan_a, 0)

        _desc_cum(hist2, ccum, zero16)
        b8 = _find_bin(ccum, zero16, zero16, TOPK)
        above8 = plsc.load_gather(ccum, [b8 + 1])

        def scan_b(i, c):
            for t in range(_U):
                ii = i * _U + t
                k = keybuf[pl.ds(ii * 16, 16)]
                c8 = lax.convert_element_type(k >> jnp.uint32(24), jnp.int32)
                f = lax.convert_element_type(
                    (k >> jnp.uint32(16)) & jnp.uint32(0xFF), jnp.int32)
                plsc.addupdate_scatter(fhist2, [lane_base + f], ones16,
                                       mask=c8 == b8)
            return c
        lax.fori_loop(0, _NV // _U, scan_b, 0)

        _desc_cum(fhist2, fcum, zero16)
        bf = _find_bin(fcum, above8, zero16, TOPK)
        thr = lax.convert_element_type(b8 * 256 + bf, jnp.uint32)

        def scan_c(i, off):
            for t in range(_U):
                ii = i * _U + t
                x = wrow[pl.ds(ii * 16, 16)]
                k = keybuf[pl.ds(ii * 16, 16)]
                m = (k >> jnp.uint32(16)) >= thr
                mi = jnp.where(m, 1, 0)
                pos = off + plsc.cumsum(mi) - mi
                okm = jnp.logical_and(m, pos < C_CAP)
                plsc.store_scatter(cvals, [pos], x, mask=okm)
                plsc.store_scatter(cidx, [pos], iota16 + ii * 16, mask=okm)
                off = off + plsc.all_reduce_population_count(m)
            return off
        lax.fori_loop(0, _NV // _U, scan_c, zero16)

        pltpu.sync_copy(cvals, vals_hbm.at[row])
        pltpu.sync_copy(cidx, cols_hbm.at[row])
        return carry0

    lax.fori_loop(0, _ROWS_PER_W, row_body, 0)


@functools.partial(
    pl.kernel,
    out_type=[jax.ShapeDtypeStruct((B, C_CAP), jnp.float32),
              jax.ShapeDtypeStruct((B, C_CAP), jnp.int32)],
    mesh=plsc.VectorSubcoreMesh(core_axis_name="c", subcore_axis_name="s"),
    compiler_params=pltpu.CompilerParams(needs_layout_passes=False),
    scratch_types=[
        pltpu.VMEM((N_MEM,), jnp.float32),
        pltpu.VMEM((N_MEM,), jnp.uint32),
        pltpu.VMEM((4096,), jnp.int32),
        pltpu.VMEM((4096,), jnp.int32),
        pltpu.VMEM((272,), jnp.int32),
        pltpu.VMEM((272,), jnp.int32),
        pltpu.VMEM((C_CAP,), jnp.float32),
        pltpu.VMEM((C_CAP,), jnp.int32),
    ],
)
def _sc_topk(w_hbm, vals_hbm, cols_hbm,
             wrow, keybuf, hist2, fhist2, ccum, fcum, cvals, cidx):
    _sc_topk_body(w_hbm, vals_hbm, cols_hbm,
                  wrow, keybuf, hist2, fhist2, ccum, fcum, cvals, cidx)


def kernel(past, abs_past, seq_start_end, end_pose, memory_past, memory_fut,
           W_np, b_np, W_ap, b_ap, W_res, b_res, W_soc,
           W_dec, b_dec, W_dec_x, b_dec_x, W_dec2, b_dec2):
    bsz = past.shape[0]
    norm_past_state = jax.nn.relu(past.reshape(bsz, -1) @ W_np + b_np)
    abs_past_state = jax.nn.relu(abs_past.reshape(bsz, -1) @ W_ap + b_ap)
    seg_id = jnp.searchsorted(seq_start_end[:, 1], jnp.arange(bsz), side='right')
    same = seg_id[:, None] == seg_id[None, :]
    d2 = ((end_pose[:, None, :] - end_pose[None, :, :]) ** 2).sum(-1)
    scores = jnp.where(same, -d2, -1e9)
    attn = jax.nn.softmax(scores, axis=1)
    abs_past_state_social = attn @ (abs_past_state @ W_soc)
    state_past = jnp.concatenate([norm_past_state, abs_past_state_social], axis=1)
    pn = _normalize(memory_past)
    sn = _normalize(state_past)
    weight_read = sn @ pn.T
    cand_vals, cand_cols = _sc_topk(weight_read)
    _, p = jax.lax.top_k(cand_vals, TOPK)
    idx = jnp.take_along_axis(cand_cols, p, axis=1)
    feat_fut = memory_fut[idx]
    nps = jnp.broadcast_to(norm_past_state[:, None, :], (bsz, TOPK, DIM))
    soc = jnp.broadcast_to(abs_past_state_social[:, None, :], (bsz, TOPK, DIM))
    input_fut = jnp.concatenate([nps, soc, feat_fut], axis=-1)
    py1 = (input_fut @ W_dec + b_dec).reshape(bsz, TOPK, FUTURE_LEN, 2)
    rx1 = (input_fut @ W_dec_x + b_dec_x).reshape(bsz, TOPK, PAST_LEN, 2)
    diff_past = past[:, None, :, :] - rx1
    diff_embed = jax.nn.relu(diff_past.reshape(bsz, TOPK, -1) @ W_res + b_res)
    state_conc = jnp.concatenate([diff_embed, soc, feat_fut], axis=-1)
    py2 = (state_conc @ W_dec2 + b_dec2).reshape(bsz, TOPK, FUTURE_LEN, 2)
    pred = py1 + py2
    pred2d = pred[:, :, 0, :]
    c = _kmeans(pred2d)
    c2 = c.reshape(bsz, NCLUSTER * 2)
    c2 = pl.pallas_call(
        _identity_kernel,
        out_shape=jax.ShapeDtypeStruct((bsz, NCLUSTER * 2), jnp.float32),
    )(c2)
    return c2.reshape(bsz, NCLUSTER, 1, 2)


# speedup vs baseline: 1.4278x; 1.4278x over previous
import functools

import functools

import jax
import jax.numpy as jnp
import numpy as np
from jax import lax
from jax.experimental import pallas as pl
from jax.experimental.pallas import tpu as pltpu
from jax.experimental.pallas import tpu_sc as plsc

PAST_LEN = 8
FUTURE_LEN = 1
DIM = 64
N_MEM = 16384
TOPK = 200
NCLUSTER = 20
KM_ITER = 10
B = 1024


def _normalize(x, eps=1e-12):
    n = jnp.linalg.norm(x, axis=1, keepdims=True)
    return x / jnp.maximum(n, eps)


def _compute_kmeans_perms():
    key = jax.random.key(123)
    keys = jax.random.split(key, KM_ITER + 1)
    outs = []
    for i in range(KM_ITER + 1):
        pki = jax.random.split(keys[i], B)
        perm = jax.vmap(lambda k: jax.random.permutation(k, TOPK))(pki)
        outs.append(np.asarray(perm[:, :NCLUSTER]))
    return np.stack(outs)


try:
    with jax.default_device(jax.devices("cpu")[0]):
        _KM_PERMS = _compute_kmeans_perms()
except Exception:
    try:
        _KM_PERMS = _compute_kmeans_perms()
    except Exception:
        _KM_PERMS = None


def _kmeans_perms():
    if _KM_PERMS is not None:
        return _KM_PERMS
    key = jax.random.key(123)
    keys = jax.random.split(key, KM_ITER + 1)
    outs = []
    for i in range(KM_ITER + 1):
        pki = jax.random.split(keys[i], B)
        perm = jax.vmap(lambda k: jax.random.permutation(k, TOPK))(pki)
        outs.append(perm[:, :NCLUSTER])
    return jnp.stack(outs)


def _kmeans(batch_x, ncluster=NCLUSTER, niter=KM_ITER):
    b, n, d = batch_x.shape
    perms = _kmeans_perms()
    idx0 = jnp.broadcast_to(jnp.asarray(perms[0])[:, :, None], (b, ncluster, d))
    c = jnp.take_along_axis(batch_x, idx0, axis=1)
    for it in range(niter):
        d2 = ((batch_x[:, :, None, :] - c[:, None, :, :]) ** 2).sum(-1)
        a = jnp.argmin(d2, axis=2)
        onehot = jax.nn.one_hot(a, ncluster, dtype=batch_x.dtype)
        counts = onehot.sum(1)
        sums = jnp.einsum('bnk,bnd->bkd', onehot, batch_x)
        cnew = sums / jnp.maximum(counts, 1e-9)[:, :, None]
        dead = counts < 0.5
        idxi = jnp.broadcast_to(jnp.asarray(perms[it + 1])[:, :, None], (b, ncluster, d))
        repl = jnp.take_along_axis(batch_x, idxi, axis=1)
        c = jnp.where(dead[:, :, None], repl, cnew)
    return c


def _identity_kernel(x_ref, o_ref):
    o_ref[...] = x_ref[...]


_SC_NC = 2
_SC_NS = 16
_NW = _SC_NC * _SC_NS
_ROWS_PER_W = B // _NW
C_CAP = 256
_NV = N_MEM // 16
_U = 8


def _keyify(x):
    ui = lax.bitcast_convert_type(x, jnp.int32)
    flip = (ui >> 31) | jnp.int32(-2147483648)
    return lax.bitcast_convert_type(ui ^ flip, jnp.uint32)


def _desc_cum(hist2_ref, cum_ref, zero16):
    carry = zero16
    for j in range(15, -1, -1):
        hv = zero16
        for l in range(16):
            hv = hv + hist2_ref[pl.ds(l * 256 + 16 * j, 16)]
        suf = plsc.cumsum(lax.rev(hv, (0,))) + carry
        cum_ref[pl.ds(16 * j, 16)] = lax.rev(suf, (0,))
        carry = carry + jnp.sum(hv)
    cum_ref[pl.ds(256, 16)] = zero16


def _find_bin(cum_ref, above, zero16, k):
    acc = zero16
    for j in range(16):
        c = cum_ref[pl.ds(16 * j, 16)]
        acc = acc + plsc.all_reduce_population_count((c + above) >= k)
    return acc - 1


def _sc_topk_body(w_hbm, vals_hbm, cols_hbm,
                  wrow, keybuf, hist2, fhist2, ccum, fcum, cvals, cidx):
    wid = lax.axis_index("s") * _SC_NC + lax.axis_index("c")
    base_row = wid * _ROWS_PER_W
    zero16 = jnp.zeros((16,), jnp.int32)
    ones16 = jnp.ones((16,), jnp.int32)
    iota16 = lax.iota(jnp.int32, 16)
    lane_base = iota16 * 256
    neginf16 = jnp.full((16,), -jnp.inf, jnp.float32)

    def row_body(r, carry0):
        row = base_row + r
        pltpu.sync_copy(w_hbm.at[row], wrow)

        @plsc.parallel_loop(0, 256, 1, unroll=_U)
        def zero_hists(j):
            hist2[pl.ds(j * 16, 16)] = zero16
            fhist2[pl.ds(j * 16, 16)] = zero16
        for j in range(C_CAP // 16):
            cvals[pl.ds(j * 16, 16)] = neginf16
            cidx[pl.ds(j * 16, 16)] = zero16

        @plsc.parallel_loop(0, _NV, 1, unroll=_U)
        def scan_a(i):
            k = _keyify(wrow[pl.ds(i * 16, 16)])
            keybuf[pl.ds(i * 16, 16)] = k
            c8 = lax.convert_element_type(k >> jnp.uint32(24), jnp.int32)
            plsc.addupdate_scatter(hist2, [lane_base + c8], ones16,
                                   mask=jnp.full((16,), True))

        _desc_cum(hist2, ccum, zero16)
        b8 = _find_bin(ccum, zero16, zero16, TOPK)
        above8 = plsc.load_gather(ccum, [b8 + 1])

        @plsc.parallel_loop(0, _NV, 1, unroll=_U)
        def scan_b(i):
            k = keybuf[pl.ds(i * 16, 16)]
            c8 = lax.convert_element_type(k >> jnp.uint32(24), jnp.int32)
            f = lax.convert_element_type(
                (k >> jnp.uint32(16)) & jnp.uint32(0xFF), jnp.int32)
            plsc.addupdate_scatter(fhist2, [lane_base + f], ones16,
                                   mask=c8 == b8)

        _desc_cum(fhist2, fcum, zero16)
        bf = _find_bin(fcum, above8, zero16, TOPK)
        thr = lax.convert_element_type(b8 * 256 + bf, jnp.uint32)

        @plsc.parallel_loop(0, _NV, 1, unroll=_U, carry=zero16)
        def scan_c(i, off):
            x = wrow[pl.ds(i * 16, 16)]
            k = keybuf[pl.ds(i * 16, 16)]
            m = (k >> jnp.uint32(16)) >= thr
            mi = jnp.where(m, 1, 0)
            pos = off + plsc.cumsum(mi) - mi
            okm = jnp.logical_and(m, pos < C_CAP)
            plsc.store_scatter(cvals, [pos], x, mask=okm)
            plsc.store_scatter(cidx, [pos], iota16 + i * 16, mask=okm)
            return off + plsc.all_reduce_population_count(m)

        pltpu.sync_copy(cvals, vals_hbm.at[row])
        pltpu.sync_copy(cidx, cols_hbm.at[row])
        return carry0

    lax.fori_loop(0, _ROWS_PER_W, row_body, 0)


@functools.partial(
    pl.kernel,
    out_type=[jax.ShapeDtypeStruct((B, C_CAP), jnp.float32),
              jax.ShapeDtypeStruct((B, C_CAP), jnp.int32)],
    mesh=plsc.VectorSubcoreMesh(core_axis_name="c", subcore_axis_name="s"),
    compiler_params=pltpu.CompilerParams(needs_layout_passes=False),
    scratch_types=[
        pltpu.VMEM((N_MEM,), jnp.float32),
        pltpu.VMEM((N_MEM,), jnp.uint32),
        pltpu.VMEM((4096,), jnp.int32),
        pltpu.VMEM((4096,), jnp.int32),
        pltpu.VMEM((272,), jnp.int32),
        pltpu.VMEM((272,), jnp.int32),
        pltpu.VMEM((C_CAP,), jnp.float32),
        pltpu.VMEM((C_CAP,), jnp.int32),
    ],
)
def _sc_topk(w_hbm, vals_hbm, cols_hbm,
             wrow, keybuf, hist2, fhist2, ccum, fcum, cvals, cidx):
    _sc_topk_body(w_hbm, vals_hbm, cols_hbm,
                  wrow, keybuf, hist2, fhist2, ccum, fcum, cvals, cidx)


def kernel(past, abs_past, seq_start_end, end_pose, memory_past, memory_fut,
           W_np, b_np, W_ap, b_ap, W_res, b_res, W_soc,
           W_dec, b_dec, W_dec_x, b_dec_x, W_dec2, b_dec2):
    bsz = past.shape[0]
    norm_past_state = jax.nn.relu(past.reshape(bsz, -1) @ W_np + b_np)
    abs_past_state = jax.nn.relu(abs_past.reshape(bsz, -1) @ W_ap + b_ap)
    seg_id = jnp.searchsorted(seq_start_end[:, 1], jnp.arange(bsz), side='right')
    same = seg_id[:, None] == seg_id[None, :]
    d2 = ((end_pose[:, None, :] - end_pose[None, :, :]) ** 2).sum(-1)
    scores = jnp.where(same, -d2, -1e9)
    attn = jax.nn.softmax(scores, axis=1)
    abs_past_state_social = attn @ (abs_past_state @ W_soc)
    state_past = jnp.concatenate([norm_past_state, abs_past_state_social], axis=1)
    pn = _normalize(memory_past)
    sn = _normalize(state_past)
    weight_read = sn @ pn.T
    cand_vals, cand_cols = _sc_topk(weight_read)
    _, p = jax.lax.top_k(cand_vals, TOPK)
    idx = jnp.take_along_axis(cand_cols, p, axis=1)
    feat_fut = memory_fut[idx]
    nps = jnp.broadcast_to(norm_past_state[:, None, :], (bsz, TOPK, DIM))
    soc = jnp.broadcast_to(abs_past_state_social[:, None, :], (bsz, TOPK, DIM))
    input_fut = jnp.concatenate([nps, soc, feat_fut], axis=-1)
    py1 = (input_fut @ W_dec + b_dec).reshape(bsz, TOPK, FUTURE_LEN, 2)
    rx1 = (input_fut @ W_dec_x + b_dec_x).reshape(bsz, TOPK, PAST_LEN, 2)
    diff_past = past[:, None, :, :] - rx1
    diff_embed = jax.nn.relu(diff_past.reshape(bsz, TOPK, -1) @ W_res + b_res)
    state_conc = jnp.concatenate([diff_embed, soc, feat_fut], axis=-1)
    py2 = (state_conc @ W_dec2 + b_dec2).reshape(bsz, TOPK, FUTURE_LEN, 2)
    pred = py1 + py2
    pred2d = pred[:, :, 0, :]
    c = _kmeans(pred2d)
    c2 = c.reshape(bsz, NCLUSTER * 2)
    c2 = pl.pallas_call(
        _identity_kernel,
        out_shape=jax.ShapeDtypeStruct((bsz, NCLUSTER * 2), jnp.float32),
    )(c2)
    return c2.reshape(bsz, NCLUSTER, 1, 2)
